# baseline (speedup 1.0000x reference)
import jax
import jax.numpy as jnp
from jax import lax
from jax.experimental import pallas as pl
from jax.experimental.pallas import tpu as pltpu
from jax.experimental.pallas import tpu_sc as plsc

F = 26
V = 100000
D = 32
B = 16384
NC, NS, L = 2, 16, 16
NW = NC * NS
ZW = F * V * D
RC = 768
KPF = V // RC
NTASK = F * KPF
BPW = B // NW
BC = 128

_params = pltpu.CompilerParams(use_tc_tiling_on_sc=True,
                               needs_layout_passes=False)


def _relayout(src, dst1d, ngrp):
    def grp(g, carry):
        rr0 = g * L
        wbase = (rr0 + lax.iota(jnp.int32, L)) * 32
        for c in range(32):
            vals = src[c, pl.ds(rr0, L)]
            plsc.store_scatter(dst1d, [wbase + c], vals)
        return carry
    lax.fori_loop(0, ngrp, grp, 0)


def _body_a(tabT, z1d, s0, s1, d0, d1, tailbuf, is0, is1, os0, os1):
    wid = lax.axis_index("s") * NC + lax.axis_index("c")

    def fk(t):
        return t // KPF, lax.rem(t, KPF)

    def in_copy(t, sb, sem):
        f, k = fk(t)
        return pltpu.make_async_copy(tabT.at[f, :, pl.ds(k * RC, RC)], sb, sem)

    def out_copy(t, db, sem):
        f, k = fk(t)
        w0 = f * (V * D) + k * (RC * D)
        return pltpu.make_async_copy(db, z1d.at[pl.ds(w0, RC * D)], sem)

    NIT = (NTASK + NW - 1) // NW

    @pl.when(wid < NTASK)
    def _():
        in_copy(wid, s0, is0).start()

    def pair(i2, carry):
        for q in (0, 1):
            ii = i2 * 2 + q
            t = wid + NW * ii
            sb, db, isem, osem = (s0, d0, is0, os0) if q == 0 else (s1, d1, is1, os1)
            nsb, nisem = (s1, is1) if q == 0 else (s0, is0)
            tp = t - 2 * NW

            @pl.when(jnp.logical_and(tp >= 0, tp < NTASK))
            def _():
                out_copy(tp, db, osem).wait()

            @pl.when(t < NTASK)
            def _():
                tn = t + NW

                @pl.when(tn < NTASK)
                def _():
                    in_copy(tn, nsb, nisem).start()
                in_copy(t, sb, isem).wait()
                _relayout(sb, db, RC // L)
                out_copy(t, db, osem).start()
        return carry

    lax.fori_loop(0, NIT // 2, pair, 0)

    for q, osem, db in ((0, os0, d0), (1, os1, d1)):
        last = wid + NW * (NIT - 2 + q)

        @pl.when(last < NTASK)
        def _():
            out_copy(last, db, osem).wait()

    @pl.when(wid < F)
    def _():
        f = wid
        pltpu.sync_copy(tabT.at[f, :, pl.ds(V - 160, 128)], s0.at[:, pl.ds(0, 128)])
        pltpu.sync_copy(tabT.at[f, :, pl.ds(V - 32, 32)], tailbuf)
        _relayout(s0, d0, 8)

        def grp(g, carry):
            rr0 = g * L
            wbase = (128 + rr0 + lax.iota(jnp.int32, L)) * 32
            for c in range(32):
                vals = tailbuf[c, pl.ds(rr0, L)]
                plsc.store_scatter(d0, [wbase + c], vals)
            return carry
        lax.fori_loop(0, 2, grp, 0)
        pltpu.sync_copy(d0.at[pl.ds(0, 160 * D)],
                        z1d.at[pl.ds(f * (V * D) + (V - 160) * D, 160 * D)])


def _stage_idx(xbufT, zidx, rem, f):
    for g in range(BC // L):
        xv = xbufT[f, pl.ds(g * L, L)]
        idx = xv + f * V
        zidx[pl.ds(g * L, L)] = lax.shift_right_logical(idx, 2)
        rem[pl.ds(g * L, L)] = lax.bitwise_and(idx, 3)


def _extract(zbuf, rem, obuf):
    for g in range(BC // L):
        iv = g * L + lax.iota(jnp.int32, L)
        zcol = rem[pl.ds(g * L, L)] * 32
        for c in range(32):
            vals = plsc.load_gather(zbuf, [iv, zcol + c])
            obuf[c, pl.ds(g * L, L)] = vals


def _body_b(xT, z_hbm, outT, xbufT, zb0, zb1, zi0, zi1, rm0, rm1,
            ob0, ob1, gs0, gs1, os0, os1):
    wid = lax.axis_index("s") * NC + lax.axis_index("c")
    b0w = wid * BPW

    def gather(zi, zb, sem):
        return pltpu.make_async_copy(z_hbm.at[zi], zb, sem)

    def out_copy(ob, f, b0, sem):
        return pltpu.make_async_copy(
            ob, outT.at[pl.ds(f * 32, 32), pl.ds(b0, BC)], sem)

    def bchunk(bc, carry):
        b0 = b0w + bc * BC
        pltpu.sync_copy(xT.at[:, pl.ds(b0, BC)], xbufT)

        _stage_idx(xbufT, zi0, rm0, 0)
        gather(zi0, zb0, gs0).start()

        def fpair(fp, c2):
            for q in (0, 1):
                f = fp * 2 + q
                zi, zb, rm, ob = (zi0, zb0, rm0, ob0) if q == 0 else (zi1, zb1, rm1, ob1)
                nzi, nzb, nrm = (zi1, zb1, rm1) if q == 0 else (zi0, zb0, rm0)
                ngs = gs1 if q == 0 else gs0
                osem = os0 if q == 0 else os1

                @pl.when(f + 1 < F)
                def _():
                    _stage_idx(xbufT, nzi, nrm, f + 1)
                    gather(nzi, nzb, ngs).start()
                gather(zi, zb, gs0 if q == 0 else gs1).wait()

                @pl.when(f >= 2)
                def _():
                    out_copy(ob, f - 2, b0, osem).wait()
                _extract(zb, rm, ob)
                out_copy(ob, f, b0, osem).start()
            return c2

        lax.fori_loop(0, F // 2, fpair, 0)
        out_copy(ob0, F - 2, b0, os0).wait()
        out_copy(ob1, F - 1, b0, os1).wait()
        return carry

    lax.fori_loop(0, BPW // BC, bchunk, 0)


@jax.jit
def _multi_embedding(xT, tabT):
    mesh = plsc.VectorSubcoreMesh(core_axis_name="c", subcore_axis_name="s")
    z1d = pl.kernel(
        _body_a, mesh=mesh, compiler_params=_params,
        out_type=jax.ShapeDtypeStruct((ZW,), jnp.float32),
        scratch_types=[
            pltpu.VMEM((32, RC), jnp.float32),
            pltpu.VMEM((32, RC), jnp.float32),
            pltpu.VMEM((RC * D,), jnp.float32),
            pltpu.VMEM((RC * D,), jnp.float32),
            pltpu.VMEM((32, 32), jnp.float32),
            pltpu.SemaphoreType.DMA,
            pltpu.SemaphoreType.DMA,
            pltpu.SemaphoreType.DMA,
            pltpu.SemaphoreType.DMA,
        ],
    )(tabT)
    z = z1d.reshape(ZW // 128, 128)
    outT = pl.kernel(
        _body_b, mesh=mesh, compiler_params=_params,
        out_type=jax.ShapeDtypeStruct((F * D, B), jnp.float32),
        scratch_types=[
            pltpu.VMEM((F, BC), jnp.int32),
            pltpu.VMEM((BC, 128), jnp.float32),
            pltpu.VMEM((BC, 128), jnp.float32),
            pltpu.VMEM((BC,), jnp.int32),
            pltpu.VMEM((BC,), jnp.int32),
            pltpu.VMEM((BC,), jnp.int32),
            pltpu.VMEM((BC,), jnp.int32),
            pltpu.VMEM((32, BC), jnp.float32),
            pltpu.VMEM((32, BC), jnp.float32),
            pltpu.SemaphoreType.DMA,
            pltpu.SemaphoreType.DMA,
            pltpu.SemaphoreType.DMA,
            pltpu.SemaphoreType.DMA,
        ],
    )(xT, z)
    return outT


def kernel(x, tables):
    tabT = jnp.transpose(tables, (0, 2, 1))
    xT = jnp.transpose(x, (1, 0))
    outT = _multi_embedding(xT, tabT)
    return jnp.transpose(outT, (1, 0))
